# Initial kernel scaffold; baseline (speedup 1.0000x reference)
import jax
import jax.numpy as jnp
from jax.experimental import pallas as pl


def kernel(x, edge_index, edge_weight, batch, W1, b1, W3, b3, W4, b4):
    raise NotImplementedError("write your pallas kernel here")


# speedup vs baseline: 8.7498x; 8.7498x over previous
import functools

import jax
import jax.numpy as jnp
from jax import lax
from jax.experimental import pallas as pl
from jax.experimental.pallas import tpu as pltpu
from jax.experimental.pallas import tpu_sc as plsc

N = 10000
E = 320000
C = 128
NUM_GRAPHS = 64

NC = 2
NS = 16
NW = NC * NS
KCH = 80
EPW = KCH * 128
EPAD = NW * EPW
NP1 = 10240
NP2 = 10240
RPT = NP2 // NS


def _mesh():
    return plsc.VectorSubcoreMesh(
        core_axis_name="c", subcore_axis_name="s", num_cores=NC, num_subcores=NS)


def _deg_body(col_hbm, ew_hbm, out_hbm, colv, ewv, acc, zbuf):
    cid = lax.axis_index("c")
    sid = lax.axis_index("s")
    wid = sid * NC + cid

    def zb(i, _):
        zbuf[pl.ds(i * 16, 16)] = jnp.zeros((16,), jnp.float32)
        return 0
    lax.fori_loop(0, 40, zb, 0)
    pltpu.sync_copy(zbuf, acc.at[pl.ds(sid * 640, 640)])
    plsc.subcore_barrier()

    pltpu.sync_copy(col_hbm.at[wid], colv)
    pltpu.sync_copy(ew_hbm.at[wid], ewv)

    def chunk(k, _):
        pltpu.sync_copy(ewv.at[k], acc.at[colv.at[k]], add=True)
        return 0
    lax.fori_loop(0, KCH, chunk, 0)
    plsc.subcore_barrier()

    pltpu.sync_copy(acc.at[pl.ds(sid * 640, 640)],
                    out_hbm.at[cid, pl.ds(sid * 640, 640)])


def _deg_call(col3, ew3):
    return pl.kernel(
        _deg_body,
        out_type=jax.ShapeDtypeStruct((NC, NP1), jnp.float32),
        mesh=_mesh(),
        scratch_types=[
            pltpu.VMEM((KCH, 128), jnp.int32),
            pltpu.VMEM((KCH, 128), jnp.float32),
            pltpu.VMEM_SHARED((NP1,), jnp.float32),
            pltpu.VMEM((640,), jnp.float32),
        ],
    )(col3, ew3)


NPH = 2
KPH = KCH // NPH


def _agg_body(row_hbm, col_hbm, ew_hbm, xs_hbm, out_hbm,
              rowv, colv, ewv, rows_v, acc, sem):
    cid = lax.axis_index("c")
    sid = lax.axis_index("s")
    wid = sid * NC + cid

    def zb(i, _):
        for j in range(8):
            rows_v[i, pl.ds(j * 16, 16)] = jnp.zeros((16,), jnp.float32)
        return 0
    lax.fori_loop(0, 128, zb, 0)
    for j in range(5):
        pltpu.sync_copy(rows_v, acc.at[pl.ds(sid * RPT + j * 128, 128)])
    plsc.subcore_barrier()

    for p in range(NPH):
        pltpu.sync_copy(row_hbm.at[wid, pl.ds(p * KPH, KPH)], rowv)
        pltpu.sync_copy(col_hbm.at[wid, pl.ds(p * KPH, KPH)], colv)
        pltpu.sync_copy(ew_hbm.at[wid, pl.ds(p * KPH, KPH)], ewv)

        def chunk(k, _):
            pltpu.async_copy(xs_hbm.at[rowv.at[k]], rows_v, sem).wait()

            def scale(e, _):
                ke = jnp.full((16,), k, jnp.int32)
                ee = jnp.full((16,), e, jnp.int32)
                w = plsc.load_gather(ewv, [ke, ee])
                for j in range(8):
                    sl = pl.ds(j * 16, 16)
                    rows_v[e, sl] = rows_v[e, sl] * w
                return 0
            lax.fori_loop(0, 128, scale, 0)

            pltpu.sync_copy(rows_v, acc.at[colv.at[k]], add=True)
            return 0
        lax.fori_loop(0, KPH, chunk, 0)
    plsc.subcore_barrier()

    pltpu.sync_copy(acc.at[pl.ds(sid * RPT, RPT)],
                    out_hbm.at[cid, pl.ds(sid * RPT, RPT)])


def _agg_call(row3, col3, ew3, xs):
    return pl.kernel(
        _agg_body,
        out_type=jax.ShapeDtypeStruct((NC, NP2, C), jnp.float32),
        mesh=_mesh(),
        scratch_types=[
            pltpu.VMEM((KPH, 128), jnp.int32),
            pltpu.VMEM((KPH, 128), jnp.int32),
            pltpu.VMEM((KPH, 128), jnp.float32),
            pltpu.VMEM((128, C), jnp.float32),
            pltpu.VMEM_SHARED((NP2, C), jnp.float32),
            pltpu.SemaphoreType.DMA,
        ],
        compiler_params=pltpu.CompilerParams(needs_layout_passes=False),
    )(row3, col3, ew3, xs)


def kernel(x, edge_index, edge_weight, batch, W1, b1, W3, b3, W4, b4):
    pad = EPAD - E
    row = jnp.concatenate([edge_index[0], jnp.zeros((pad,), jnp.int32)])
    col = jnp.concatenate([edge_index[1], jnp.zeros((pad,), jnp.int32)])
    ew = jnp.concatenate([edge_weight, jnp.zeros((pad,), jnp.float32)])
    row3 = row.reshape(NW, KCH, 128)
    col3 = col.reshape(NW, KCH, 128)
    ew3 = ew.reshape(NW, KCH, 128)

    deg_parts = _deg_call(col3, ew3)
    deg = deg_parts[0, :N] + deg_parts[1, :N] + 1.0

    dis = jnp.where(deg > 0, lax.rsqrt(deg), 0.0)
    xs = x * dis[:, None]

    def spass(v):
        parts = _agg_call(row3, col3, ew3, v)
        return parts[0, :N] + parts[1, :N]

    ax = dis[:, None] * (spass(xs) + dis[:, None] * x)
    h = jax.nn.relu(ax @ W1 + b1)
    hs = h * dis[:, None]
    ah = dis[:, None] * (spass(hs) + dis[:, None] * h)
    embed = ah @ W3 + b3
    g = ah @ W4 + b4
    graph_embed = jax.ops.segment_sum(g, batch, num_segments=NUM_GRAPHS)
    return (embed, graph_embed)


# speedup vs baseline: 9.1897x; 1.0503x over previous
import functools

import jax
import jax.numpy as jnp
from jax import lax
from jax.experimental import pallas as pl
from jax.experimental.pallas import tpu as pltpu
from jax.experimental.pallas import tpu_sc as plsc

N = 10000
E = 320000
C = 128
NUM_GRAPHS = 64

NC = 2
NS = 16
NW = NC * NS
KCH = 80
EPW = KCH * 128
EPAD = NW * EPW
NP1 = 10240
NP2 = 10240
RPT = NP2 // NS


def _mesh():
    return plsc.VectorSubcoreMesh(
        core_axis_name="c", subcore_axis_name="s", num_cores=NC, num_subcores=NS)


def _deg_body(col_hbm, ew_hbm, out_hbm, colv, ewv, acc, zbuf):
    cid = lax.axis_index("c")
    sid = lax.axis_index("s")
    wid = sid * NC + cid

    def zb(i, _):
        zbuf[pl.ds(i * 16, 16)] = jnp.zeros((16,), jnp.float32)
        return 0
    lax.fori_loop(0, 40, zb, 0)
    pltpu.sync_copy(zbuf, acc.at[pl.ds(sid * 640, 640)])
    plsc.subcore_barrier()

    pltpu.sync_copy(col_hbm.at[wid], colv)
    pltpu.sync_copy(ew_hbm.at[wid], ewv)

    def chunk(k, _):
        pltpu.sync_copy(ewv.at[k], acc.at[colv.at[k]], add=True)
        return 0
    lax.fori_loop(0, KCH, chunk, 0)
    plsc.subcore_barrier()

    pltpu.sync_copy(acc.at[pl.ds(sid * 640, 640)],
                    out_hbm.at[cid, pl.ds(sid * 640, 640)])


def _deg_call(col3, ew3):
    return pl.kernel(
        _deg_body,
        out_type=jax.ShapeDtypeStruct((NC, NP1), jnp.float32),
        mesh=_mesh(),
        scratch_types=[
            pltpu.VMEM((KCH, 128), jnp.int32),
            pltpu.VMEM((KCH, 128), jnp.float32),
            pltpu.VMEM_SHARED((NP1,), jnp.float32),
            pltpu.VMEM((640,), jnp.float32),
        ],
    )(col3, ew3)


NPH = 2
KPH = KCH // NPH


def _agg_body(row_hbm, col_hbm, ew_hbm, xs_hbm, out_hbm,
              rowv, colv, ewv, rows_v, acc, sem):
    cid = lax.axis_index("c")
    sid = lax.axis_index("s")
    wid = sid * NC + cid

    def zb(i, _):
        for j in range(8):
            rows_v[i, pl.ds(j * 16, 16)] = jnp.zeros((16,), jnp.float32)
        return 0
    lax.fori_loop(0, 128, zb, 0)
    for j in range(5):
        pltpu.sync_copy(rows_v, acc.at[pl.ds(sid * RPT + j * 128, 128)])
    plsc.subcore_barrier()

    for p in range(NPH):
        pltpu.sync_copy(row_hbm.at[wid, pl.ds(p * KPH, KPH)], rowv)
        pltpu.sync_copy(col_hbm.at[wid, pl.ds(p * KPH, KPH)], colv)
        pltpu.sync_copy(ew_hbm.at[wid, pl.ds(p * KPH, KPH)], ewv)

        def chunk(k, _):
            pltpu.async_copy(xs_hbm.at[rowv.at[k]], rows_v, sem).wait()

            def scale(e, _):
                ke = jnp.full((16,), k, jnp.int32)
                ee = jnp.full((16,), e, jnp.int32)
                w = plsc.load_gather(ewv, [ke, ee])
                for j in range(8):
                    sl = pl.ds(j * 16, 16)
                    rows_v[e, sl] = rows_v[e, sl] * w
                return 0
            lax.fori_loop(0, 128, scale, 0)

            pltpu.sync_copy(rows_v, acc.at[colv.at[k]], add=True)
            return 0
        lax.fori_loop(0, KPH, chunk, 0)
    plsc.subcore_barrier()

    pltpu.sync_copy(acc.at[pl.ds(sid * RPT, RPT)],
                    out_hbm.at[cid, pl.ds(sid * RPT, RPT)])


def _agg_call(row3, col3, ew3, xs):
    return pl.kernel(
        _agg_body,
        out_type=jax.ShapeDtypeStruct((NC, NP2, C), jnp.float32),
        mesh=_mesh(),
        scratch_types=[
            pltpu.VMEM((KPH, 128), jnp.int32),
            pltpu.VMEM((KPH, 128), jnp.int32),
            pltpu.VMEM((KPH, 128), jnp.float32),
            pltpu.VMEM((128, C), jnp.float32),
            pltpu.VMEM_SHARED((NP2, C), jnp.float32),
            pltpu.SemaphoreType.DMA,
        ],
        compiler_params=pltpu.CompilerParams(needs_layout_passes=False),
    )(row3, col3, ew3, xs)


BR = 1000
GSTEPS = N // BR


def _norm_body(deg2_ref, x_ref, dis_ref, xs_ref):
    deg = deg2_ref[:, 0:1] + deg2_ref[:, 1:2] + 1.0
    dis = jnp.where(deg > 0, lax.rsqrt(deg), 0.0)
    dis_ref[...] = dis
    xs_ref[...] = x_ref[...] * dis


def _norm_call(deg2, x):
    return pl.pallas_call(
        _norm_body,
        grid=(GSTEPS,),
        in_specs=[
            pl.BlockSpec((BR, 2), lambda i: (i, 0)),
            pl.BlockSpec((BR, C), lambda i: (i, 0)),
        ],
        out_specs=[
            pl.BlockSpec((BR, 1), lambda i: (i, 0)),
            pl.BlockSpec((BR, C), lambda i: (i, 0)),
        ],
        out_shape=[
            jax.ShapeDtypeStruct((N, 1), jnp.float32),
            jax.ShapeDtypeStruct((N, C), jnp.float32),
        ],
    )(deg2, x)


def _layer1_body(t_ref, x_ref, dis_ref, w_ref, b_ref, h_ref, hs_ref):
    dis = dis_ref[...]
    ax = dis * (t_ref[0] + t_ref[1] + dis * x_ref[...])
    h = jnp.maximum(jnp.dot(ax, w_ref[...],
                            preferred_element_type=jnp.float32) + b_ref[...],
                    0.0)
    h_ref[...] = h
    hs_ref[...] = h * dis


def _layer1_call(t_parts, x, dis, W1, b1):
    return pl.pallas_call(
        _layer1_body,
        grid=(GSTEPS,),
        in_specs=[
            pl.BlockSpec((NC, BR, C), lambda i: (0, i, 0)),
            pl.BlockSpec((BR, C), lambda i: (i, 0)),
            pl.BlockSpec((BR, 1), lambda i: (i, 0)),
            pl.BlockSpec((C, C), lambda i: (0, 0)),
            pl.BlockSpec((1, C), lambda i: (0, 0)),
        ],
        out_specs=[
            pl.BlockSpec((BR, C), lambda i: (i, 0)),
            pl.BlockSpec((BR, C), lambda i: (i, 0)),
        ],
        out_shape=[
            jax.ShapeDtypeStruct((N, C), jnp.float32),
            jax.ShapeDtypeStruct((N, C), jnp.float32),
        ],
    )(t_parts, x, dis, W1, b1)


def _layer2_body(t_ref, h_ref, dis_ref, w3_ref, b3_ref, w4_ref, b4_ref,
                 batch_ref, embed_ref, gemb_ref):
    i = pl.program_id(0)
    dis = dis_ref[...]
    ah = dis * (t_ref[0] + t_ref[1] + dis * h_ref[...])
    embed_ref[...] = jnp.dot(ah, w3_ref[...],
                             preferred_element_type=jnp.float32) + b3_ref[...]
    g = jnp.dot(ah, w4_ref[...],
                preferred_element_type=jnp.float32) + b4_ref[...]
    iota = lax.broadcasted_iota(jnp.int32, (BR, NUM_GRAPHS), 1)
    onehot = (batch_ref[...] == iota).astype(jnp.float32)
    contrib = lax.dot_general(onehot, g, (((0,), (0,)), ((), ())),
                              preferred_element_type=jnp.float32)

    @pl.when(i == 0)
    def _():
        gemb_ref[...] = jnp.zeros_like(gemb_ref)
    gemb_ref[...] += contrib


def _layer2_call(t_parts, h, dis, W3, b3, W4, b4, batch1):
    return pl.pallas_call(
        _layer2_body,
        grid=(GSTEPS,),
        in_specs=[
            pl.BlockSpec((NC, BR, C), lambda i: (0, i, 0)),
            pl.BlockSpec((BR, C), lambda i: (i, 0)),
            pl.BlockSpec((BR, 1), lambda i: (i, 0)),
            pl.BlockSpec((C, C), lambda i: (0, 0)),
            pl.BlockSpec((1, C), lambda i: (0, 0)),
            pl.BlockSpec((C, C), lambda i: (0, 0)),
            pl.BlockSpec((1, C), lambda i: (0, 0)),
            pl.BlockSpec((BR, 1), lambda i: (i, 0)),
        ],
        out_specs=[
            pl.BlockSpec((BR, C), lambda i: (i, 0)),
            pl.BlockSpec((NUM_GRAPHS, C), lambda i: (0, 0)),
        ],
        out_shape=[
            jax.ShapeDtypeStruct((N, C), jnp.float32),
            jax.ShapeDtypeStruct((NUM_GRAPHS, C), jnp.float32),
        ],
    )(t_parts, h, dis, W3, b3, W4, b4, batch1)


def kernel(x, edge_index, edge_weight, batch, W1, b1, W3, b3, W4, b4):
    pad = EPAD - E
    row = jnp.concatenate([edge_index[0], jnp.zeros((pad,), jnp.int32)])
    col = jnp.concatenate([edge_index[1], jnp.zeros((pad,), jnp.int32)])
    ew = jnp.concatenate([edge_weight, jnp.zeros((pad,), jnp.float32)])
    row3 = row.reshape(NW, KCH, 128)
    col3 = col.reshape(NW, KCH, 128)
    ew3 = ew.reshape(NW, KCH, 128)

    deg_parts = _deg_call(col3, ew3)
    deg2 = deg_parts[:, :N].T
    dis, xs = _norm_call(deg2, x)

    t1 = _agg_call(row3, col3, ew3, xs)
    h, hs = _layer1_call(t1, x, dis, W1, b1.reshape(1, C))

    t2 = _agg_call(row3, col3, ew3, hs)
    embed, graph_embed = _layer2_call(
        t2, h, dis, W3, b3.reshape(1, C), W4, b4.reshape(1, C),
        batch.reshape(N, 1))
    return (embed, graph_embed)


# speedup vs baseline: 10.7577x; 1.1706x over previous
import functools

import jax
import jax.numpy as jnp
from jax import lax
from jax.experimental import pallas as pl
from jax.experimental.pallas import tpu as pltpu
from jax.experimental.pallas import tpu_sc as plsc

N = 10000
E = 320000
C = 128
NUM_GRAPHS = 64

NC = 2
NS = 16
NW = NC * NS
KCH = 80
EPW = KCH * 128
EPAD = NW * EPW
NP1 = 10240
NP2 = 10240
RPT = NP2 // NS


def _mesh():
    return plsc.VectorSubcoreMesh(
        core_axis_name="c", subcore_axis_name="s", num_cores=NC, num_subcores=NS)


def _deg_body(col_hbm, ew_hbm, out_hbm, colv, ewv, acc, zbuf):
    cid = lax.axis_index("c")
    sid = lax.axis_index("s")
    wid = sid * NC + cid

    def zb(i, _):
        zbuf[pl.ds(i * 16, 16)] = jnp.zeros((16,), jnp.float32)
        return 0
    lax.fori_loop(0, 40, zb, 0)
    pltpu.sync_copy(zbuf, acc.at[pl.ds(sid * 640, 640)])
    plsc.subcore_barrier()

    pltpu.sync_copy(col_hbm.at[wid], colv)
    pltpu.sync_copy(ew_hbm.at[wid], ewv)

    def chunk(k, _):
        pltpu.sync_copy(ewv.at[k], acc.at[colv.at[k]], add=True)
        return 0
    lax.fori_loop(0, KCH, chunk, 0)
    plsc.subcore_barrier()

    pltpu.sync_copy(acc.at[pl.ds(sid * 640, 640)],
                    out_hbm.at[cid, pl.ds(sid * 640, 640)])


def _deg_call(col3, ew3):
    return pl.kernel(
        _deg_body,
        out_type=jax.ShapeDtypeStruct((NC, NP1), jnp.float32),
        mesh=_mesh(),
        scratch_types=[
            pltpu.VMEM((KCH, 128), jnp.int32),
            pltpu.VMEM((KCH, 128), jnp.float32),
            pltpu.VMEM_SHARED((NP1,), jnp.float32),
            pltpu.VMEM((640,), jnp.float32),
        ],
    )(col3, ew3)


NPH = 2
KPH = KCH // NPH


def _agg_body(row_hbm, col_hbm, ew_hbm, xs_hbm, out_hbm,
              rowv, colv, ewv, rows_v, rows_w, acc,
              gsem_a, gsem_b, ssem_a, ssem_b):
    cid = lax.axis_index("c")
    sid = lax.axis_index("s")
    wid = sid * NC + cid

    def zb(i, _):
        for j in range(8):
            rows_v[i, pl.ds(j * 16, 16)] = jnp.zeros((16,), jnp.float32)
        return 0
    lax.fori_loop(0, 128, zb, 0)
    for j in range(5):
        pltpu.sync_copy(rows_v, acc.at[pl.ds(sid * RPT + j * 128, 128)])
    plsc.subcore_barrier()

    bufs = (rows_v, rows_w)
    gsems = (gsem_a, gsem_b)
    ssems = (ssem_a, ssem_b)

    def scale(buf, k, e, _):
        ke = jnp.full((16,), k, jnp.int32)
        ee = jnp.full((16,), e, jnp.int32)
        w = plsc.load_gather(ewv, [ke, ee])
        for j in range(8):
            sl = pl.ds(j * 16, 16)
            buf[e, sl] = buf[e, sl] * w
        return 0

    for p in range(NPH):
        pltpu.sync_copy(row_hbm.at[wid, pl.ds(p * KPH, KPH)], rowv)
        pltpu.sync_copy(col_hbm.at[wid, pl.ds(p * KPH, KPH)], colv)
        pltpu.sync_copy(ew_hbm.at[wid, pl.ds(p * KPH, KPH)], ewv)

        pltpu.async_copy(xs_hbm.at[rowv.at[0]], bufs[0], gsems[0])

        def pair(k2, _):
            ka = 2 * k2
            kb = ka + 1
            pltpu.make_async_copy(xs_hbm.at[rowv.at[ka]], bufs[0],
                                  gsems[0]).wait()

            @pl.when(k2 > 0)
            def _():
                pltpu.make_async_copy(bufs[1], acc.at[colv.at[ka]],
                                      ssems[1]).wait()
            pltpu.async_copy(xs_hbm.at[rowv.at[kb]], bufs[1], gsems[1])
            lax.fori_loop(0, 128, functools.partial(scale, bufs[0], ka), 0,
                          unroll=8)
            pltpu.async_copy(bufs[0], acc.at[colv.at[ka]], ssems[0], add=True)

            pltpu.make_async_copy(xs_hbm.at[rowv.at[kb]], bufs[1],
                                  gsems[1]).wait()
            lax.fori_loop(0, 128, functools.partial(scale, bufs[1], kb), 0,
                          unroll=8)
            pltpu.async_copy(bufs[1], acc.at[colv.at[kb]], ssems[1], add=True)

            @pl.when(ka + 2 < KPH)
            def _():
                pltpu.make_async_copy(bufs[0], acc.at[colv.at[ka]],
                                      ssems[0]).wait()
                pltpu.async_copy(xs_hbm.at[rowv.at[ka + 2]], bufs[0],
                                 gsems[0])
            return 0
        lax.fori_loop(0, KPH // 2, pair, 0)

        pltpu.make_async_copy(bufs[0], acc.at[colv.at[0]], ssems[0]).wait()
        pltpu.make_async_copy(bufs[1], acc.at[colv.at[0]], ssems[1]).wait()
    plsc.subcore_barrier()

    pltpu.sync_copy(acc.at[pl.ds(sid * RPT, RPT)],
                    out_hbm.at[cid, pl.ds(sid * RPT, RPT)])


def _agg_call(row3, col3, ew3, xs):
    return pl.kernel(
        _agg_body,
        out_type=jax.ShapeDtypeStruct((NC, NP2, C), jnp.float32),
        mesh=_mesh(),
        scratch_types=[
            pltpu.VMEM((KPH, 128), jnp.int32),
            pltpu.VMEM((KPH, 128), jnp.int32),
            pltpu.VMEM((KPH, 128), jnp.float32),
            pltpu.VMEM((128, C), jnp.float32),
            pltpu.VMEM((128, C), jnp.float32),
            pltpu.VMEM_SHARED((NP2, C), jnp.float32),
            pltpu.SemaphoreType.DMA,
            pltpu.SemaphoreType.DMA,
            pltpu.SemaphoreType.DMA,
            pltpu.SemaphoreType.DMA,
        ],
        compiler_params=pltpu.CompilerParams(needs_layout_passes=False),
    )(row3, col3, ew3, xs)


BR = 1000
GSTEPS = N // BR


def _norm_body(deg2_ref, x_ref, dis_ref, xs_ref):
    deg = deg2_ref[:, 0:1] + deg2_ref[:, 1:2] + 1.0
    dis = jnp.where(deg > 0, lax.rsqrt(deg), 0.0)
    dis_ref[...] = dis
    xs_ref[...] = x_ref[...] * dis


def _norm_call(deg2, x):
    return pl.pallas_call(
        _norm_body,
        grid=(GSTEPS,),
        in_specs=[
            pl.BlockSpec((BR, 2), lambda i: (i, 0)),
            pl.BlockSpec((BR, C), lambda i: (i, 0)),
        ],
        out_specs=[
            pl.BlockSpec((BR, 1), lambda i: (i, 0)),
            pl.BlockSpec((BR, C), lambda i: (i, 0)),
        ],
        out_shape=[
            jax.ShapeDtypeStruct((N, 1), jnp.float32),
            jax.ShapeDtypeStruct((N, C), jnp.float32),
        ],
    )(deg2, x)


def _layer1_body(t_ref, x_ref, dis_ref, w_ref, b_ref, h_ref, hs_ref):
    dis = dis_ref[...]
    ax = dis * (t_ref[0] + t_ref[1] + dis * x_ref[...])
    h = jnp.maximum(jnp.dot(ax, w_ref[...],
                            preferred_element_type=jnp.float32) + b_ref[...],
                    0.0)
    h_ref[...] = h
    hs_ref[...] = h * dis


def _layer1_call(t_parts, x, dis, W1, b1):
    return pl.pallas_call(
        _layer1_body,
        grid=(GSTEPS,),
        in_specs=[
            pl.BlockSpec((NC, BR, C), lambda i: (0, i, 0)),
            pl.BlockSpec((BR, C), lambda i: (i, 0)),
            pl.BlockSpec((BR, 1), lambda i: (i, 0)),
            pl.BlockSpec((C, C), lambda i: (0, 0)),
            pl.BlockSpec((1, C), lambda i: (0, 0)),
        ],
        out_specs=[
            pl.BlockSpec((BR, C), lambda i: (i, 0)),
            pl.BlockSpec((BR, C), lambda i: (i, 0)),
        ],
        out_shape=[
            jax.ShapeDtypeStruct((N, C), jnp.float32),
            jax.ShapeDtypeStruct((N, C), jnp.float32),
        ],
    )(t_parts, x, dis, W1, b1)


def _layer2_body(t_ref, h_ref, dis_ref, w3_ref, b3_ref, w4_ref, b4_ref,
                 batch_ref, embed_ref, gemb_ref):
    i = pl.program_id(0)
    dis = dis_ref[...]
    ah = dis * (t_ref[0] + t_ref[1] + dis * h_ref[...])
    embed_ref[...] = jnp.dot(ah, w3_ref[...],
                             preferred_element_type=jnp.float32) + b3_ref[...]
    g = jnp.dot(ah, w4_ref[...],
                preferred_element_type=jnp.float32) + b4_ref[...]
    iota = lax.broadcasted_iota(jnp.int32, (BR, NUM_GRAPHS), 1)
    onehot = (batch_ref[...] == iota).astype(jnp.float32)
    contrib = lax.dot_general(onehot, g, (((0,), (0,)), ((), ())),
                              preferred_element_type=jnp.float32)

    @pl.when(i == 0)
    def _():
        gemb_ref[...] = jnp.zeros_like(gemb_ref)
    gemb_ref[...] += contrib


def _layer2_call(t_parts, h, dis, W3, b3, W4, b4, batch1):
    return pl.pallas_call(
        _layer2_body,
        grid=(GSTEPS,),
        in_specs=[
            pl.BlockSpec((NC, BR, C), lambda i: (0, i, 0)),
            pl.BlockSpec((BR, C), lambda i: (i, 0)),
            pl.BlockSpec((BR, 1), lambda i: (i, 0)),
            pl.BlockSpec((C, C), lambda i: (0, 0)),
            pl.BlockSpec((1, C), lambda i: (0, 0)),
            pl.BlockSpec((C, C), lambda i: (0, 0)),
            pl.BlockSpec((1, C), lambda i: (0, 0)),
            pl.BlockSpec((BR, 1), lambda i: (i, 0)),
        ],
        out_specs=[
            pl.BlockSpec((BR, C), lambda i: (i, 0)),
            pl.BlockSpec((NUM_GRAPHS, C), lambda i: (0, 0)),
        ],
        out_shape=[
            jax.ShapeDtypeStruct((N, C), jnp.float32),
            jax.ShapeDtypeStruct((NUM_GRAPHS, C), jnp.float32),
        ],
    )(t_parts, h, dis, W3, b3, W4, b4, batch1)


def kernel(x, edge_index, edge_weight, batch, W1, b1, W3, b3, W4, b4):
    pad = EPAD - E
    row = jnp.concatenate([edge_index[0], jnp.zeros((pad,), jnp.int32)])
    col = jnp.concatenate([edge_index[1], jnp.zeros((pad,), jnp.int32)])
    ew = jnp.concatenate([edge_weight, jnp.zeros((pad,), jnp.float32)])
    row3 = row.reshape(NW, KCH, 128)
    col3 = col.reshape(NW, KCH, 128)
    ew3 = ew.reshape(NW, KCH, 128)

    deg_parts = _deg_call(col3, ew3)
    deg2 = deg_parts[:, :N].T
    dis, xs = _norm_call(deg2, x)

    t1 = _agg_call(row3, col3, ew3, xs)
    h, hs = _layer1_call(t1, x, dis, W1, b1.reshape(1, C))

    t2 = _agg_call(row3, col3, ew3, hs)
    embed, graph_embed = _layer2_call(
        t2, h, dis, W3, b3.reshape(1, C), W4, b4.reshape(1, C),
        batch.reshape(N, 1))
    return (embed, graph_embed)
